# Initial kernel scaffold; baseline (speedup 1.0000x reference)
import functools

import jax
import jax.numpy as jnp
from jax import lax
from jax.experimental import pallas as pl
from jax.experimental.pallas import tpu as pltpu
from jax.experimental.pallas import tpu_sc as plsc

_R = 32
_V = _R * _R * _R
_B = 4
_C = 128
_N = 100000
_CH = 4000
_NCHUNK = _N // _CH


def _coords_body(coords_ref, nc_ref, idx_ref):
    x = coords_ref[0]
    mean = jnp.mean(x, axis=1, keepdims=True)
    c = x - mean
    norm2 = jnp.sum(c * c, axis=0, keepdims=True)
    denom = jnp.max(jnp.sqrt(norm2))
    denom = jnp.maximum(denom * 2.0, 1e-6)
    nc = jnp.clip((c / denom + 0.5) * _R, 0.0, _R - 1.0)
    nc_ref[0] = nc
    vox = jnp.round(nc).astype(jnp.int32)
    idx = vox[0:1] * (_R * _R) + vox[1:2] * _R + vox[2:3]
    idx_ref[...] = idx


def _compute_idx(coords):
    return pl.pallas_call(
        _coords_body,
        grid=(_B,),
        in_specs=[pl.BlockSpec((1, 3, _N), lambda b: (b, 0, 0))],
        out_specs=[
            pl.BlockSpec((1, 3, _N), lambda b: (b, 0, 0)),
            pl.BlockSpec((1, _N), lambda b: (b, 0)),
        ],
        out_shape=[
            jax.ShapeDtypeStruct((_B, 3, _N), jnp.float32),
            jax.ShapeDtypeStruct((_B, _N), jnp.int32),
        ],
    )(coords)


def _scatter_body(feat_hbm, idx_hbm, out_hbm,
                  acc_a, acc_b, recip_b, idx_buf, f_a, f_b, shared_recip):
    cid = lax.axis_index("c")
    sid = lax.axis_index("s")
    my_batch = sid // 4
    cbase = cid * 64 + (sid % 4) * 16

    zeros16 = jnp.zeros((16,), jnp.float32)
    ones16 = jnp.full((16,), 1.0, jnp.float32)

    @pl.when(sid < _B)
    def _():
        def zb(i, carry):
            acc_a[pl.ds(i * 16, 16)] = zeros16
            return carry
        lax.fori_loop(0, _V // 16, zb, None)

        def cb(k, carry):
            pltpu.sync_copy(idx_hbm.at[sid, pl.ds(k * _CH, _CH)], idx_buf)

            def gb(g, c2):
                iv = idx_buf[pl.ds(g * 16, 16)]
                plsc.addupdate_scatter(acc_a, [iv], ones16)
                return c2
            lax.fori_loop(0, _CH // 16, gb, None)
            return carry
        lax.fori_loop(0, _NCHUNK, cb, None)

        def rb(i, carry):
            s = pl.ds(i * 16, 16)
            acc_a[s] = 1.0 / jnp.maximum(acc_a[s], 1.0)
            return carry
        lax.fori_loop(0, _V // 16, rb, None)
        pltpu.sync_copy(acc_a, shared_recip.at[sid])

    plsc.subcore_barrier()
    pltpu.sync_copy(shared_recip.at[my_batch], recip_b)

    def pair_body(p, carry):
        c0 = cbase + 2 * p
        c1 = c0 + 1

        def zb(i, c2):
            s = pl.ds(i * 16, 16)
            acc_a[s] = zeros16
            acc_b[s] = zeros16
            return c2
        lax.fori_loop(0, _V // 16, zb, None)

        def cb(k, c2):
            off = k * _CH
            pltpu.sync_copy(idx_hbm.at[my_batch, pl.ds(off, _CH)], idx_buf)
            pltpu.sync_copy(feat_hbm.at[my_batch, c0, pl.ds(off, _CH)], f_a)
            pltpu.sync_copy(feat_hbm.at[my_batch, c1, pl.ds(off, _CH)], f_b)

            def gb(g, c3):
                s = pl.ds(g * 16, 16)
                iv = idx_buf[s]
                plsc.addupdate_scatter(acc_a, [iv], f_a[s])
                plsc.addupdate_scatter(acc_b, [iv], f_b[s])
                return c3
            lax.fori_loop(0, _CH // 16, gb, None)
            return c2
        lax.fori_loop(0, _NCHUNK, cb, None)

        def nb(i, c2):
            s = pl.ds(i * 16, 16)
            r = recip_b[s]
            acc_a[s] = acc_a[s] * r
            acc_b[s] = acc_b[s] * r
            return c2
        lax.fori_loop(0, _V // 16, nb, None)
        pltpu.sync_copy(acc_a, out_hbm.at[my_batch, c0])
        pltpu.sync_copy(acc_b, out_hbm.at[my_batch, c1])
        return carry
    lax.fori_loop(0, 8, pair_body, None)


_scatter_call = functools.partial(
    pl.kernel,
    out_type=jax.ShapeDtypeStruct((_B, _C, _V), jnp.float32),
    mesh=plsc.VectorSubcoreMesh(core_axis_name="c", subcore_axis_name="s"),
    scratch_types=[
        pltpu.VMEM((_V,), jnp.float32),
        pltpu.VMEM((_V,), jnp.float32),
        pltpu.VMEM((_V,), jnp.float32),
        pltpu.VMEM((_CH,), jnp.int32),
        pltpu.VMEM((_CH,), jnp.float32),
        pltpu.VMEM((_CH,), jnp.float32),
        pltpu.VMEM_SHARED((_B, _V), jnp.float32),
    ],
)(_scatter_body)


def kernel(features, coords):
    nc, idx = _compute_idx(coords)
    out = _scatter_call(features, idx)
    return out.reshape(_B, _C, _R, _R, _R), nc


# speedup vs baseline: 1.0217x; 1.0217x over previous
import functools

import jax
import jax.numpy as jnp
from jax import lax
from jax.experimental import pallas as pl
from jax.experimental.pallas import tpu as pltpu
from jax.experimental.pallas import tpu_sc as plsc

_R = 32
_V = _R * _R * _R
_B = 4
_C = 128
_N = 100000
_CH = 4000
_NCHUNK = _N // _CH


def _coords_body(coords_ref, nc_ref, idx_ref):
    x = coords_ref[0]
    mean = jnp.mean(x, axis=1, keepdims=True)
    c = x - mean
    norm2 = jnp.sum(c * c, axis=0, keepdims=True)
    denom = jnp.max(jnp.sqrt(norm2))
    denom = jnp.maximum(denom * 2.0, 1e-6)
    nc = jnp.clip((c / denom + 0.5) * _R, 0.0, _R - 1.0)
    nc_ref[0] = nc
    vox = jnp.round(nc).astype(jnp.int32)
    idx = vox[0:1] * (_R * _R) + vox[1:2] * _R + vox[2:3]
    idx_ref[0] = idx


def _compute_idx(coords):
    return pl.pallas_call(
        _coords_body,
        grid=(_B,),
        in_specs=[pl.BlockSpec((1, 3, _N), lambda b: (b, 0, 0))],
        out_specs=[
            pl.BlockSpec((1, 3, _N), lambda b: (b, 0, 0)),
            pl.BlockSpec((1, 1, _N), lambda b: (b, 0, 0)),
        ],
        out_shape=[
            jax.ShapeDtypeStruct((_B, 3, _N), jnp.float32),
            jax.ShapeDtypeStruct((_B, 1, _N), jnp.int32),
        ],
    )(coords)


def _scatter_body(feat_hbm, idx_hbm, out_hbm,
                  acc_a, acc_b, recip_b, idx_buf, f_a, f_b, shared_recip):
    cid = lax.axis_index("c")
    sid = lax.axis_index("s")
    my_batch = sid // 4
    cbase = cid * 64 + (sid % 4) * 16

    zeros16 = jnp.zeros((16,), jnp.float32)
    ones16 = jnp.full((16,), 1.0, jnp.float32)

    @pl.when(sid < _B)
    def _():
        def zb(i, carry):
            acc_a[pl.ds(i * 16, 16)] = zeros16
            return carry
        lax.fori_loop(0, _V // 16, zb, None)

        def cb(k, carry):
            pltpu.sync_copy(idx_hbm.at[sid, pl.ds(k * _CH, _CH)], idx_buf)

            def gb(g, c2):
                iv = idx_buf[pl.ds(g * 16, 16)]
                plsc.addupdate_scatter(acc_a, [iv], ones16)
                return c2
            lax.fori_loop(0, _CH // 16, gb, None)
            return carry
        lax.fori_loop(0, _NCHUNK, cb, None)

        def rb(i, carry):
            s = pl.ds(i * 16, 16)
            acc_a[s] = 1.0 / jnp.maximum(acc_a[s], 1.0)
            return carry
        lax.fori_loop(0, _V // 16, rb, None)
        pltpu.sync_copy(acc_a, shared_recip.at[sid])

    plsc.subcore_barrier()
    pltpu.sync_copy(shared_recip.at[my_batch], recip_b)

    def pair_body(p, carry):
        c0 = cbase + 2 * p
        c1 = c0 + 1

        def zb(i, c2):
            s = pl.ds(i * 16, 16)
            acc_a[s] = zeros16
            acc_b[s] = zeros16
            return c2
        lax.fori_loop(0, _V // 16, zb, None)

        def cb(k, c2):
            off = k * _CH
            pltpu.sync_copy(idx_hbm.at[my_batch, pl.ds(off, _CH)], idx_buf)
            pltpu.sync_copy(feat_hbm.at[my_batch, c0, pl.ds(off, _CH)], f_a)
            pltpu.sync_copy(feat_hbm.at[my_batch, c1, pl.ds(off, _CH)], f_b)

            def gb(g, c3):
                s = pl.ds(g * 16, 16)
                iv = idx_buf[s]
                plsc.addupdate_scatter(acc_a, [iv], f_a[s])
                plsc.addupdate_scatter(acc_b, [iv], f_b[s])
                return c3
            lax.fori_loop(0, _CH // 16, gb, None)
            return c2
        lax.fori_loop(0, _NCHUNK, cb, None)

        def nb(i, c2):
            s = pl.ds(i * 16, 16)
            r = recip_b[s]
            acc_a[s] = acc_a[s] * r
            acc_b[s] = acc_b[s] * r
            return c2
        lax.fori_loop(0, _V // 16, nb, None)
        pltpu.sync_copy(acc_a, out_hbm.at[my_batch, c0])
        pltpu.sync_copy(acc_b, out_hbm.at[my_batch, c1])
        return carry
    lax.fori_loop(0, 8, pair_body, None)


_scatter_call = functools.partial(
    pl.kernel,
    out_type=jax.ShapeDtypeStruct((_B, _C, _V), jnp.float32),
    mesh=plsc.VectorSubcoreMesh(core_axis_name="c", subcore_axis_name="s",
                                num_cores=2, num_subcores=16),
    compiler_params=pltpu.CompilerParams(use_tc_tiling_on_sc=False,
                                         needs_layout_passes=False),
    scratch_types=[
        pltpu.VMEM((_V,), jnp.float32),
        pltpu.VMEM((_V,), jnp.float32),
        pltpu.VMEM((_V,), jnp.float32),
        pltpu.VMEM((_CH,), jnp.int32),
        pltpu.VMEM((_CH,), jnp.float32),
        pltpu.VMEM((_CH,), jnp.float32),
        pltpu.VMEM_SHARED((_B, _V), jnp.float32),
    ],
)(_scatter_body)


def kernel(features, coords):
    nc, idx = _compute_idx(coords)
    out = _scatter_call(features, idx.reshape(_B, _N))
    return out.reshape(_B, _C, _R, _R, _R), nc


# speedup vs baseline: 1.4700x; 1.4388x over previous
import functools

import jax
import jax.numpy as jnp
from jax import lax
from jax.experimental import pallas as pl
from jax.experimental.pallas import tpu as pltpu
from jax.experimental.pallas import tpu_sc as plsc

_R = 32
_V = _R * _R * _R
_B = 4
_C = 128
_N = 100000
_CH = 4000
_NCHUNK = _N // _CH


def _coords_body(coords_ref, nc_ref, idx_ref):
    x = coords_ref[0]
    mean = jnp.mean(x, axis=1, keepdims=True)
    c = x - mean
    norm2 = jnp.sum(c * c, axis=0, keepdims=True)
    denom = jnp.max(jnp.sqrt(norm2))
    denom = jnp.maximum(denom * 2.0, 1e-6)
    nc = jnp.clip((c / denom + 0.5) * _R, 0.0, _R - 1.0)
    nc_ref[0] = nc
    vox = jnp.round(nc).astype(jnp.int32)
    idx = vox[0:1] * (_R * _R) + vox[1:2] * _R + vox[2:3]
    idx_ref[0] = idx


def _compute_idx(coords):
    return pl.pallas_call(
        _coords_body,
        grid=(_B,),
        in_specs=[pl.BlockSpec((1, 3, _N), lambda b: (b, 0, 0))],
        out_specs=[
            pl.BlockSpec((1, 3, _N), lambda b: (b, 0, 0)),
            pl.BlockSpec((1, 1, _N), lambda b: (b, 0, 0)),
        ],
        out_shape=[
            jax.ShapeDtypeStruct((_B, 3, _N), jnp.float32),
            jax.ShapeDtypeStruct((_B, 1, _N), jnp.int32),
        ],
    )(coords)


def _scatter_body(feat_hbm, idx_hbm, out_hbm,
                  acc_a, acc_b, recip_b, idx_bufs, f_a, f_b,
                  sem0, sem1, shared_recip):
    cid = lax.axis_index("c")
    sid = lax.axis_index("s")
    my_batch = sid // 4
    cbase = cid * 64 + (sid % 4) * 16

    zeros16 = jnp.zeros((16,), jnp.float32)
    ones16 = jnp.full((16,), 1.0, jnp.float32)
    sems = (sem0, sem1)

    def start_idx(batch, k, par):
        pltpu.async_copy(idx_hbm.at[batch, pl.ds(k * _CH, _CH)],
                         idx_bufs.at[par], sems[par])

    def drain_idx(par):
        pltpu.make_async_copy(idx_hbm.at[0, pl.ds(0, _CH)],
                              idx_bufs.at[par], sems[par]).wait()

    def start_feat(c0, c1, k, par):
        off = k * _CH
        pltpu.async_copy(feat_hbm.at[my_batch, c0, pl.ds(off, _CH)],
                         f_a.at[par], sems[par])
        pltpu.async_copy(feat_hbm.at[my_batch, c1, pl.ds(off, _CH)],
                         f_b.at[par], sems[par])

    def drain_feat(par):
        pltpu.make_async_copy(feat_hbm.at[0, 0, pl.ds(0, _CH)],
                              f_a.at[par], sems[par]).wait()
        pltpu.make_async_copy(feat_hbm.at[0, 0, pl.ds(0, _CH)],
                              f_b.at[par], sems[par]).wait()

    @pl.when(sid < _B)
    def _():
        def zb(i, carry):
            for u in range(8):
                acc_a[pl.ds((i * 8 + u) * 16, 16)] = zeros16
            return carry
        lax.fori_loop(0, _V // 128, zb, None)

        start_idx(sid, 0, 0)
        start_idx(sid, 1, 1)

        def cb(kk, carry):
            for par in range(2):
                k = kk * 2 + par

                @pl.when(k < _NCHUNK)
                def _():
                    drain_idx(par)

                    def gb(g, c3):
                        for u in range(10):
                            s = pl.ds((g * 10 + u) * 16, 16)
                            iv = idx_bufs.at[par][s]
                            plsc.addupdate_scatter(acc_a, [iv], ones16)
                        return c3
                    lax.fori_loop(0, _CH // 160, gb, None)

                    @pl.when(k + 2 < _NCHUNK)
                    def _():
                        start_idx(sid, k + 2, par)
            return carry
        lax.fori_loop(0, (_NCHUNK + 2) // 2, cb, None)

        def rb(i, carry):
            for u in range(4):
                s = pl.ds((i * 4 + u) * 16, 16)
                acc_a[s] = 1.0 / jnp.maximum(acc_a[s], 1.0)
            return carry
        lax.fori_loop(0, _V // 64, rb, None)
        pltpu.sync_copy(acc_a, shared_recip.at[sid])

    plsc.subcore_barrier()
    pltpu.sync_copy(shared_recip.at[my_batch], recip_b)

    def pair_body(p, carry):
        c0 = cbase + 2 * p
        c1 = c0 + 1

        def zb(i, c2):
            for u in range(8):
                s = pl.ds((i * 8 + u) * 16, 16)
                acc_a[s] = zeros16
                acc_b[s] = zeros16
            return c2
        lax.fori_loop(0, _V // 128, zb, None)

        start_idx(my_batch, 0, 0)
        start_feat(c0, c1, 0, 0)
        start_idx(my_batch, 1, 1)
        start_feat(c0, c1, 1, 1)

        def cb(kk, c2):
            for par in range(2):
                k = kk * 2 + par

                @pl.when(k < _NCHUNK)
                def _():
                    drain_idx(par)
                    drain_feat(par)

                    def gb(g, c3):
                        for u in range(5):
                            s = pl.ds((g * 5 + u) * 16, 16)
                            iv = idx_bufs.at[par][s]
                            plsc.addupdate_scatter(acc_a, [iv], f_a.at[par][s])
                            plsc.addupdate_scatter(acc_b, [iv], f_b.at[par][s])
                        return c3
                    lax.fori_loop(0, _CH // 80, gb, None)

                    @pl.when(k + 2 < _NCHUNK)
                    def _():
                        start_idx(my_batch, k + 2, par)
                        start_feat(c0, c1, k + 2, par)
            return c2
        lax.fori_loop(0, (_NCHUNK + 2) // 2, cb, None)

        def nb(i, c2):
            for u in range(4):
                s = pl.ds((i * 4 + u) * 16, 16)
                r = recip_b[s]
                acc_a[s] = acc_a[s] * r
                acc_b[s] = acc_b[s] * r
            return c2
        lax.fori_loop(0, _V // 64, nb, None)
        pltpu.sync_copy(acc_a, out_hbm.at[my_batch, c0])
        pltpu.sync_copy(acc_b, out_hbm.at[my_batch, c1])
        return carry
    lax.fori_loop(0, 8, pair_body, None)


_scatter_call = functools.partial(
    pl.kernel,
    out_type=jax.ShapeDtypeStruct((_B, _C, _V), jnp.float32),
    mesh=plsc.VectorSubcoreMesh(core_axis_name="c", subcore_axis_name="s",
                                num_cores=2, num_subcores=16),
    compiler_params=pltpu.CompilerParams(use_tc_tiling_on_sc=False,
                                         needs_layout_passes=False),
    scratch_types=[
        pltpu.VMEM((_V,), jnp.float32),
        pltpu.VMEM((_V,), jnp.float32),
        pltpu.VMEM((_V,), jnp.float32),
        pltpu.VMEM((2, _CH), jnp.int32),
        pltpu.VMEM((2, _CH), jnp.float32),
        pltpu.VMEM((2, _CH), jnp.float32),
        pltpu.SemaphoreType.DMA,
        pltpu.SemaphoreType.DMA,
        pltpu.VMEM_SHARED((_B, _V), jnp.float32),
    ],
)(_scatter_body)


def kernel(features, coords):
    nc, idx = _compute_idx(coords)
    out = _scatter_call(features, idx.reshape(_B, _N))
    return out.reshape(_B, _C, _R, _R, _R), nc


# speedup vs baseline: 1.4714x; 1.0009x over previous
import functools

import jax
import jax.numpy as jnp
from jax import lax
from jax.experimental import pallas as pl
from jax.experimental.pallas import tpu as pltpu
from jax.experimental.pallas import tpu_sc as plsc

_R = 32
_V = _R * _R * _R
_B = 4
_C = 128
_N = 100000
_CH = 4000
_NCHUNK = _N // _CH
_NPAD = 100352


def _coords_body(coords_ref, nc_ref, idx_ref):
    x = coords_ref[0]
    mean = jnp.mean(x, axis=1, keepdims=True)
    c = x - mean
    norm2 = jnp.sum(c * c, axis=0, keepdims=True)
    denom = jnp.max(jnp.sqrt(norm2))
    denom = jnp.maximum(denom * 2.0, 1e-6)
    nc = jnp.clip((c / denom + 0.5) * _R, 0.0, _R - 1.0)
    nc_ref[0] = nc
    vox = jnp.round(nc).astype(jnp.int32)
    idx = vox[0:1] * (_R * _R) + vox[1:2] * _R + vox[2:3]
    idx_ref[pl.ds(0, _N)] = idx.reshape(_N)


def _compute_idx(coords):
    return pl.pallas_call(
        _coords_body,
        grid=(_B,),
        in_specs=[pl.BlockSpec((1, 3, _N), lambda b: (b, 0, 0))],
        out_specs=[
            pl.BlockSpec((1, 3, _N), lambda b: (b, 0, 0)),
            pl.BlockSpec((_NPAD,), lambda b: (b,)),
        ],
        out_shape=[
            jax.ShapeDtypeStruct((_B, 3, _N), jnp.float32),
            jax.ShapeDtypeStruct((_B * _NPAD,), jnp.int32),
        ],
    )(coords)


def _scatter_body(feat_hbm, idx_hbm, out_hbm,
                  acc_a, acc_b, recip_b, idx_bufs, f_a, f_b,
                  sem0, sem1, shared_recip):
    cid = lax.axis_index("c")
    sid = lax.axis_index("s")
    my_batch = sid // 4
    cbase = cid * 64 + (sid % 4) * 16

    zeros16 = jnp.zeros((16,), jnp.float32)
    ones16 = jnp.full((16,), 1.0, jnp.float32)
    sems = (sem0, sem1)

    def start_idx(batch, k, par):
        pltpu.async_copy(idx_hbm.at[pl.ds(batch * _NPAD + k * _CH, _CH)],
                         idx_bufs.at[par], sems[par])

    def drain_idx(par):
        pltpu.make_async_copy(idx_hbm.at[pl.ds(0, _CH)],
                              idx_bufs.at[par], sems[par]).wait()

    def start_feat(c0, c1, k, par):
        off = k * _CH
        pltpu.async_copy(feat_hbm.at[my_batch, c0, pl.ds(off, _CH)],
                         f_a.at[par], sems[par])
        pltpu.async_copy(feat_hbm.at[my_batch, c1, pl.ds(off, _CH)],
                         f_b.at[par], sems[par])

    def drain_feat(par):
        pltpu.make_async_copy(feat_hbm.at[0, 0, pl.ds(0, _CH)],
                              f_a.at[par], sems[par]).wait()
        pltpu.make_async_copy(feat_hbm.at[0, 0, pl.ds(0, _CH)],
                              f_b.at[par], sems[par]).wait()

    @pl.when(sid < _B)
    def _():
        def zb(i, carry):
            for u in range(8):
                acc_a[pl.ds((i * 8 + u) * 16, 16)] = zeros16
            return carry
        lax.fori_loop(0, _V // 128, zb, None)

        start_idx(sid, 0, 0)
        start_idx(sid, 1, 1)

        def cb(kk, carry):
            for par in range(2):
                k = kk * 2 + par

                @pl.when(k < _NCHUNK)
                def _():
                    drain_idx(par)

                    def gb(g, c3):
                        for u in range(10):
                            s = pl.ds((g * 10 + u) * 16, 16)
                            iv = idx_bufs.at[par][s]
                            plsc.addupdate_scatter(acc_a, [iv], ones16)
                        return c3
                    lax.fori_loop(0, _CH // 160, gb, None)

                    @pl.when(k + 2 < _NCHUNK)
                    def _():
                        start_idx(sid, k + 2, par)
            return carry
        lax.fori_loop(0, (_NCHUNK + 2) // 2, cb, None)

        def rb(i, carry):
            for u in range(4):
                s = pl.ds((i * 4 + u) * 16, 16)
                acc_a[s] = 1.0 / jnp.maximum(acc_a[s], 1.0)
            return carry
        lax.fori_loop(0, _V // 64, rb, None)
        pltpu.sync_copy(acc_a, shared_recip.at[sid])

    plsc.subcore_barrier()
    pltpu.sync_copy(shared_recip.at[my_batch], recip_b)

    def pair_body(p, carry):
        c0 = cbase + 2 * p
        c1 = c0 + 1

        def zb(i, c2):
            for u in range(8):
                s = pl.ds((i * 8 + u) * 16, 16)
                acc_a[s] = zeros16
                acc_b[s] = zeros16
            return c2
        lax.fori_loop(0, _V // 128, zb, None)

        start_idx(my_batch, 0, 0)
        start_feat(c0, c1, 0, 0)
        start_idx(my_batch, 1, 1)
        start_feat(c0, c1, 1, 1)

        def cb(kk, c2):
            for par in range(2):
                k = kk * 2 + par

                @pl.when(k < _NCHUNK)
                def _():
                    drain_idx(par)
                    drain_feat(par)

                    def gb(g, c3):
                        for u in range(5):
                            s = pl.ds((g * 5 + u) * 16, 16)
                            iv = idx_bufs.at[par][s]
                            plsc.addupdate_scatter(acc_a, [iv], f_a.at[par][s])
                            plsc.addupdate_scatter(acc_b, [iv], f_b.at[par][s])
                        return c3
                    lax.fori_loop(0, _CH // 80, gb, None)

                    @pl.when(k + 2 < _NCHUNK)
                    def _():
                        start_idx(my_batch, k + 2, par)
                        start_feat(c0, c1, k + 2, par)
            return c2
        lax.fori_loop(0, (_NCHUNK + 2) // 2, cb, None)

        def nb(i, c2):
            for u in range(4):
                s = pl.ds((i * 4 + u) * 16, 16)
                r = recip_b[s]
                acc_a[s] = acc_a[s] * r
                acc_b[s] = acc_b[s] * r
            return c2
        lax.fori_loop(0, _V // 64, nb, None)
        pltpu.sync_copy(acc_a, out_hbm.at[my_batch, c0])
        pltpu.sync_copy(acc_b, out_hbm.at[my_batch, c1])
        return carry
    lax.fori_loop(0, 8, pair_body, None)


_scatter_call = functools.partial(
    pl.kernel,
    out_type=jax.ShapeDtypeStruct((_B, _C, _V), jnp.float32),
    mesh=plsc.VectorSubcoreMesh(core_axis_name="c", subcore_axis_name="s",
                                num_cores=2, num_subcores=16),
    compiler_params=pltpu.CompilerParams(use_tc_tiling_on_sc=False,
                                         needs_layout_passes=False),
    scratch_types=[
        pltpu.VMEM((_V,), jnp.float32),
        pltpu.VMEM((_V,), jnp.float32),
        pltpu.VMEM((_V,), jnp.float32),
        pltpu.VMEM((2, _CH), jnp.int32),
        pltpu.VMEM((2, _CH), jnp.float32),
        pltpu.VMEM((2, _CH), jnp.float32),
        pltpu.SemaphoreType.DMA,
        pltpu.SemaphoreType.DMA,
        pltpu.VMEM_SHARED((_B, _V), jnp.float32),
    ],
)(_scatter_body)


def kernel(features, coords):
    nc, idx = _compute_idx(coords)
    out = _scatter_call(features, idx)
    return out.reshape(_B, _C, _R, _R, _R), nc


# speedup vs baseline: 1.9431x; 1.3206x over previous
import functools

import jax
import jax.numpy as jnp
from jax import lax
from jax.experimental import pallas as pl
from jax.experimental.pallas import tpu as pltpu
from jax.experimental.pallas import tpu_sc as plsc

_R = 32
_V = _R * _R * _R
_B = 4
_C = 128
_N = 100000
_CH = 4000
_NCHUNK = _N // _CH
_NPAD = 100352


def _coords_body(coords_ref, nc_ref, idx_ref):
    x = coords_ref[0]
    mean = jnp.mean(x, axis=1, keepdims=True)
    c = x - mean
    norm2 = jnp.sum(c * c, axis=0, keepdims=True)
    denom = jnp.max(jnp.sqrt(norm2))
    denom = jnp.maximum(denom * 2.0, 1e-6)
    nc = jnp.clip((c / denom + 0.5) * _R, 0.0, _R - 1.0)
    nc_ref[0] = nc
    vox = jnp.round(nc).astype(jnp.int32)
    idx = vox[0:1] * (_R * _R) + vox[1:2] * _R + vox[2:3]
    idx_ref[pl.ds(0, _N)] = idx.reshape(_N)


def _compute_idx(coords):
    return pl.pallas_call(
        _coords_body,
        grid=(_B,),
        in_specs=[pl.BlockSpec((1, 3, _N), lambda b: (b, 0, 0))],
        out_specs=[
            pl.BlockSpec((1, 3, _N), lambda b: (b, 0, 0)),
            pl.BlockSpec((_NPAD,), lambda b: (b,)),
        ],
        out_shape=[
            jax.ShapeDtypeStruct((_B, 3, _N), jnp.float32),
            jax.ShapeDtypeStruct((_B * _NPAD,), jnp.int32),
        ],
    )(coords)


def _scatter_body(feat_hbm, idx_hbm, out_hbm,
                  acc_a, acc_b, recip_b, idx_bufs, f_a, f_b,
                  sem0, sem1, shared_recip):
    cid = lax.axis_index("c")
    sid = lax.axis_index("s")
    my_batch = sid // 4
    cbase = cid * 64 + (sid % 4) * 16

    zeros16 = jnp.zeros((16,), jnp.float32)
    ones16 = jnp.full((16,), 1.0, jnp.float32)
    sems = (sem0, sem1)

    def start_idx(batch, k, par):
        pltpu.async_copy(idx_hbm.at[pl.ds(batch * _NPAD + k * _CH, _CH)],
                         idx_bufs.at[par], sems[par])

    def drain_idx(par):
        pltpu.make_async_copy(idx_hbm.at[pl.ds(0, _CH)],
                              idx_bufs.at[par], sems[par]).wait()

    def start_feat(c0, c1, k, par):
        off = k * _CH
        pltpu.async_copy(feat_hbm.at[my_batch, c0, pl.ds(off, _CH)],
                         f_a.at[par], sems[par])
        pltpu.async_copy(feat_hbm.at[my_batch, c1, pl.ds(off, _CH)],
                         f_b.at[par], sems[par])

    def drain_feat(par):
        pltpu.make_async_copy(feat_hbm.at[0, 0, pl.ds(0, _CH)],
                              f_a.at[par], sems[par]).wait()
        pltpu.make_async_copy(feat_hbm.at[0, 0, pl.ds(0, _CH)],
                              f_b.at[par], sems[par]).wait()

    @pl.when(sid < _B)
    def _():
        def zb(i, carry):
            for u in range(8):
                acc_a[pl.ds((i * 8 + u) * 16, 16)] = zeros16
            return carry
        lax.fori_loop(0, _V // 128, zb, None)

        start_idx(sid, 0, 0)
        start_idx(sid, 1, 1)

        def cb(kk, carry):
            for par in range(2):
                k = kk * 2 + par

                @pl.when(k < _NCHUNK)
                def _():
                    drain_idx(par)

                    def gb(g, c3):
                        ivs = [idx_bufs.at[par][pl.ds((g * 10 + u) * 16, 16)]
                               for u in range(10)]
                        for u in range(10):
                            plsc.addupdate_scatter(acc_a, [ivs[u]], ones16)
                        return c3
                    lax.fori_loop(0, _CH // 160, gb, None)

                    @pl.when(k + 2 < _NCHUNK)
                    def _():
                        start_idx(sid, k + 2, par)
            return carry
        lax.fori_loop(0, (_NCHUNK + 2) // 2, cb, None)

        def rb(i, carry):
            slcs = [pl.ds((i * 4 + u) * 16, 16) for u in range(4)]
            vs = [acc_a[s] for s in slcs]
            for u in range(4):
                acc_a[slcs[u]] = 1.0 / jnp.maximum(vs[u], 1.0)
            return carry
        lax.fori_loop(0, _V // 64, rb, None)
        pltpu.sync_copy(acc_a, shared_recip.at[sid])

    plsc.subcore_barrier()
    pltpu.sync_copy(shared_recip.at[my_batch], recip_b)

    def pair_body(p, carry):
        c0 = cbase + 2 * p
        c1 = c0 + 1

        def zb(i, c2):
            for u in range(8):
                s = pl.ds((i * 8 + u) * 16, 16)
                acc_a[s] = zeros16
                acc_b[s] = zeros16
            return c2
        lax.fori_loop(0, _V // 128, zb, None)

        start_idx(my_batch, 0, 0)
        start_feat(c0, c1, 0, 0)
        start_idx(my_batch, 1, 1)
        start_feat(c0, c1, 1, 1)

        def cb(kk, c2):
            for par in range(2):
                k = kk * 2 + par

                @pl.when(k < _NCHUNK)
                def _():
                    drain_idx(par)
                    drain_feat(par)

                    def gb(g, c3):
                        slcs = [pl.ds((g * 5 + u) * 16, 16) for u in range(5)]
                        ivs = [idx_bufs.at[par][s] for s in slcs]
                        vas = [f_a.at[par][s] for s in slcs]
                        vbs = [f_b.at[par][s] for s in slcs]
                        for u in range(5):
                            plsc.addupdate_scatter(acc_a, [ivs[u]], vas[u])
                            plsc.addupdate_scatter(acc_b, [ivs[u]], vbs[u])
                        return c3
                    lax.fori_loop(0, _CH // 80, gb, None)

                    @pl.when(k + 2 < _NCHUNK)
                    def _():
                        start_idx(my_batch, k + 2, par)
                        start_feat(c0, c1, k + 2, par)
            return c2
        lax.fori_loop(0, (_NCHUNK + 2) // 2, cb, None)

        def nb(i, c2):
            slcs = [pl.ds((i * 4 + u) * 16, 16) for u in range(4)]
            rs = [recip_b[s] for s in slcs]
            avs = [acc_a[s] for s in slcs]
            bvs = [acc_b[s] for s in slcs]
            for u in range(4):
                acc_a[slcs[u]] = avs[u] * rs[u]
                acc_b[slcs[u]] = bvs[u] * rs[u]
            return c2
        lax.fori_loop(0, _V // 64, nb, None)
        pltpu.sync_copy(acc_a, out_hbm.at[my_batch, c0])
        pltpu.sync_copy(acc_b, out_hbm.at[my_batch, c1])
        return carry
    lax.fori_loop(0, 8, pair_body, None)


_scatter_call = functools.partial(
    pl.kernel,
    out_type=jax.ShapeDtypeStruct((_B, _C, _V), jnp.float32),
    mesh=plsc.VectorSubcoreMesh(core_axis_name="c", subcore_axis_name="s",
                                num_cores=2, num_subcores=16),
    compiler_params=pltpu.CompilerParams(use_tc_tiling_on_sc=False,
                                         needs_layout_passes=False),
    scratch_types=[
        pltpu.VMEM((_V,), jnp.float32),
        pltpu.VMEM((_V,), jnp.float32),
        pltpu.VMEM((_V,), jnp.float32),
        pltpu.VMEM((2, _CH), jnp.int32),
        pltpu.VMEM((2, _CH), jnp.float32),
        pltpu.VMEM((2, _CH), jnp.float32),
        pltpu.SemaphoreType.DMA,
        pltpu.SemaphoreType.DMA,
        pltpu.VMEM_SHARED((_B, _V), jnp.float32),
    ],
)(_scatter_body)


def kernel(features, coords):
    nc, idx = _compute_idx(coords)
    out = _scatter_call(features, idx)
    return out.reshape(_B, _C, _R, _R, _R), nc


# speedup vs baseline: 1.9963x; 1.0273x over previous
import functools

import jax
import jax.numpy as jnp
from jax import lax
from jax.experimental import pallas as pl
from jax.experimental.pallas import tpu as pltpu
from jax.experimental.pallas import tpu_sc as plsc

_R = 32
_V = _R * _R * _R
_B = 4
_C = 128
_N = 100000
_CH = 4000
_NCHUNK = _N // _CH
_NPAD = 100352

_MESH = plsc.VectorSubcoreMesh(core_axis_name="c", subcore_axis_name="s",
                               num_cores=2, num_subcores=16)
_SC_PARAMS = pltpu.CompilerParams(use_tc_tiling_on_sc=False,
                                  needs_layout_passes=False)


def _coords_body(coords_ref, nc_ref, idx_ref):
    x = coords_ref[0]
    mean = jnp.mean(x, axis=1, keepdims=True)
    c = x - mean
    norm2 = jnp.sum(c * c, axis=0, keepdims=True)
    denom = jnp.max(jnp.sqrt(norm2))
    denom = jnp.maximum(denom * 2.0, 1e-6)
    nc = jnp.clip((c / denom + 0.5) * _R, 0.0, _R - 1.0)
    nc_ref[0] = nc
    vox = jnp.round(nc).astype(jnp.int32)
    idx = vox[0:1] * (_R * _R) + vox[1:2] * _R + vox[2:3]
    idx_ref[pl.ds(0, _N)] = idx.reshape(_N)


def _compute_idx(coords):
    return pl.pallas_call(
        _coords_body,
        grid=(_B,),
        in_specs=[pl.BlockSpec((1, 3, _N), lambda b: (b, 0, 0))],
        out_specs=[
            pl.BlockSpec((1, 3, _N), lambda b: (b, 0, 0)),
            pl.BlockSpec((_NPAD,), lambda b: (b,)),
        ],
        out_shape=[
            jax.ShapeDtypeStruct((_B, 3, _N), jnp.float32),
            jax.ShapeDtypeStruct((_B * _NPAD,), jnp.int32),
        ],
    )(coords)


def _counts_body(idx_hbm, recip_hbm, acc, idx_bufs, sem0, sem1):
    cid = lax.axis_index("c")
    sid = lax.axis_index("s")
    sems = (sem0, sem1)
    zeros16 = jnp.zeros((16,), jnp.float32)
    ones16 = jnp.full((16,), 1.0, jnp.float32)

    def start_idx(batch, k, par):
        pltpu.async_copy(idx_hbm.at[pl.ds(batch * _NPAD + k * _CH, _CH)],
                         idx_bufs.at[par], sems[par])

    def drain_idx(par):
        pltpu.make_async_copy(idx_hbm.at[pl.ds(0, _CH)],
                              idx_bufs.at[par], sems[par]).wait()

    @pl.when(jnp.logical_and(cid == 0, sid < _B))
    def _():
        def zb(i, carry):
            for u in range(8):
                acc[pl.ds((i * 8 + u) * 16, 16)] = zeros16
            return carry
        lax.fori_loop(0, _V // 128, zb, None)

        start_idx(sid, 0, 0)
        start_idx(sid, 1, 1)

        def cb(kk, carry):
            for par in range(2):
                k = kk * 2 + par

                @pl.when(k < _NCHUNK)
                def _():
                    drain_idx(par)

                    def gb(g, c3):
                        ivs = [idx_bufs.at[par][pl.ds((g * 10 + u) * 16, 16)]
                               for u in range(10)]
                        for u in range(10):
                            plsc.addupdate_scatter(acc, [ivs[u]], ones16)
                        return c3
                    lax.fori_loop(0, _CH // 160, gb, None)

                    @pl.when(k + 2 < _NCHUNK)
                    def _():
                        start_idx(sid, k + 2, par)
            return carry
        lax.fori_loop(0, (_NCHUNK + 2) // 2, cb, None)

        def rb(i, carry):
            slcs = [pl.ds((i * 4 + u) * 16, 16) for u in range(4)]
            vs = [acc[s] for s in slcs]
            for u in range(4):
                acc[slcs[u]] = 1.0 / jnp.maximum(vs[u], 1.0)
            return carry
        lax.fori_loop(0, _V // 64, rb, None)
        pltpu.sync_copy(acc, recip_hbm.at[pl.ds(sid * _V, _V)])


_counts_call = functools.partial(
    pl.kernel,
    out_type=jax.ShapeDtypeStruct((_B * _V,), jnp.float32),
    mesh=_MESH,
    compiler_params=_SC_PARAMS,
    scratch_types=[
        pltpu.VMEM((_V,), jnp.float32),
        pltpu.VMEM((2, _CH), jnp.int32),
        pltpu.SemaphoreType.DMA,
        pltpu.SemaphoreType.DMA,
    ],
)(_counts_body)


def _scatter_body(feat_hbm, idx_hbm, recip_hbm, out_hbm,
                  acc_a, acc_b, recip_b, idx_bufs, f_a, f_b, sem0, sem1):
    cid = lax.axis_index("c")
    sid = lax.axis_index("s")
    my_batch = sid // 4
    cbase = cid * 64 + (sid % 4) * 16

    zeros16 = jnp.zeros((16,), jnp.float32)
    sems = (sem0, sem1)

    def start_idx(k, par):
        pltpu.async_copy(idx_hbm.at[pl.ds(my_batch * _NPAD + k * _CH, _CH)],
                         idx_bufs.at[par], sems[par])

    def drain_idx(par):
        pltpu.make_async_copy(idx_hbm.at[pl.ds(0, _CH)],
                              idx_bufs.at[par], sems[par]).wait()

    def start_feat(c0, c1, k, par):
        off = k * _CH
        pltpu.async_copy(feat_hbm.at[my_batch, c0, pl.ds(off, _CH)],
                         f_a.at[par], sems[par])
        pltpu.async_copy(feat_hbm.at[my_batch, c1, pl.ds(off, _CH)],
                         f_b.at[par], sems[par])

    def drain_feat(par):
        pltpu.make_async_copy(feat_hbm.at[0, 0, pl.ds(0, _CH)],
                              f_a.at[par], sems[par]).wait()
        pltpu.make_async_copy(feat_hbm.at[0, 0, pl.ds(0, _CH)],
                              f_b.at[par], sems[par]).wait()

    pltpu.sync_copy(recip_hbm.at[pl.ds(my_batch * _V, _V)], recip_b)

    def pair_body(p, carry):
        c0 = cbase + 2 * p
        c1 = c0 + 1

        def zb(i, c2):
            for u in range(8):
                s = pl.ds((i * 8 + u) * 16, 16)
                acc_a[s] = zeros16
                acc_b[s] = zeros16
            return c2
        lax.fori_loop(0, _V // 128, zb, None)

        start_idx(0, 0)
        start_feat(c0, c1, 0, 0)
        start_idx(1, 1)
        start_feat(c0, c1, 1, 1)

        def cb(kk, c2):
            for par in range(2):
                k = kk * 2 + par

                @pl.when(k < _NCHUNK)
                def _():
                    drain_idx(par)
                    drain_feat(par)

                    def gb(g, c3):
                        slcs = [pl.ds((g * 5 + u) * 16, 16) for u in range(5)]
                        ivs = [idx_bufs.at[par][s] for s in slcs]
                        vas = [f_a.at[par][s] for s in slcs]
                        vbs = [f_b.at[par][s] for s in slcs]
                        for u in range(5):
                            plsc.addupdate_scatter(acc_a, [ivs[u]], vas[u])
                            plsc.addupdate_scatter(acc_b, [ivs[u]], vbs[u])
                        return c3
                    lax.fori_loop(0, _CH // 80, gb, None)

                    @pl.when(k + 2 < _NCHUNK)
                    def _():
                        start_idx(k + 2, par)
                        start_feat(c0, c1, k + 2, par)
            return c2
        lax.fori_loop(0, (_NCHUNK + 2) // 2, cb, None)

        def nb(i, c2):
            slcs = [pl.ds((i * 4 + u) * 16, 16) for u in range(4)]
            rs = [recip_b[s] for s in slcs]
            avs = [acc_a[s] for s in slcs]
            bvs = [acc_b[s] for s in slcs]
            for u in range(4):
                acc_a[slcs[u]] = avs[u] * rs[u]
                acc_b[slcs[u]] = bvs[u] * rs[u]
            return c2
        lax.fori_loop(0, _V // 64, nb, None)
        pltpu.sync_copy(acc_a, out_hbm.at[my_batch, c0])
        pltpu.sync_copy(acc_b, out_hbm.at[my_batch, c1])
        return carry
    lax.fori_loop(0, 8, pair_body, None)


_scatter_call = functools.partial(
    pl.kernel,
    out_type=jax.ShapeDtypeStruct((_B, _C, _V), jnp.float32),
    mesh=_MESH,
    compiler_params=_SC_PARAMS,
    scratch_types=[
        pltpu.VMEM((_V,), jnp.float32),
        pltpu.VMEM((_V,), jnp.float32),
        pltpu.VMEM((_V,), jnp.float32),
        pltpu.VMEM((2, _CH), jnp.int32),
        pltpu.VMEM((2, _CH), jnp.float32),
        pltpu.VMEM((2, _CH), jnp.float32),
        pltpu.SemaphoreType.DMA,
        pltpu.SemaphoreType.DMA,
    ],
)(_scatter_body)


def kernel(features, coords):
    nc, idx = _compute_idx(coords)
    recip = _counts_call(idx)
    out = _scatter_call(features, idx, recip)
    return out.reshape(_B, _C, _R, _R, _R), nc


# speedup vs baseline: 2.1967x; 1.1004x over previous
import functools

import jax
import jax.numpy as jnp
from jax import lax
from jax.experimental import pallas as pl
from jax.experimental.pallas import tpu as pltpu
from jax.experimental.pallas import tpu_sc as plsc

_R = 32
_V = _R * _R * _R
_B = 4
_C = 128
_N = 100000
_CH = 4000
_NCHUNK = _N // _CH
_NPAD = 100352

_MESH = plsc.VectorSubcoreMesh(core_axis_name="c", subcore_axis_name="s",
                               num_cores=2, num_subcores=16)
_SC_PARAMS = pltpu.CompilerParams(use_tc_tiling_on_sc=False,
                                  needs_layout_passes=False)


def _coords_body(coords_ref, nc_ref, idx_ref):
    x = coords_ref[0]
    mean = jnp.mean(x, axis=1, keepdims=True)
    c = x - mean
    norm2 = jnp.sum(c * c, axis=0, keepdims=True)
    denom = jnp.max(jnp.sqrt(norm2))
    denom = jnp.maximum(denom * 2.0, 1e-6)
    nc = jnp.clip((c / denom + 0.5) * _R, 0.0, _R - 1.0)
    nc_ref[0] = nc
    vox = jnp.round(nc).astype(jnp.int32)
    idx = vox[0:1] * (_R * _R) + vox[1:2] * _R + vox[2:3]
    idx_ref[pl.ds(0, _N)] = idx.reshape(_N)


def _compute_idx(coords):
    return pl.pallas_call(
        _coords_body,
        grid=(_B,),
        in_specs=[pl.BlockSpec((1, 3, _N), lambda b: (b, 0, 0))],
        out_specs=[
            pl.BlockSpec((1, 3, _N), lambda b: (b, 0, 0)),
            pl.BlockSpec((_NPAD,), lambda b: (b,)),
        ],
        out_shape=[
            jax.ShapeDtypeStruct((_B, 3, _N), jnp.float32),
            jax.ShapeDtypeStruct((_B * _NPAD,), jnp.int32),
        ],
    )(coords)


def _counts_body(idx_hbm, recip_hbm, acc, idx_bufs, sem0, sem1):
    cid = lax.axis_index("c")
    sid = lax.axis_index("s")
    sems = (sem0, sem1)
    zeros16 = jnp.zeros((16,), jnp.float32)
    ones16 = jnp.full((16,), 1.0, jnp.float32)

    def start_idx(batch, k, par):
        pltpu.async_copy(idx_hbm.at[pl.ds(batch * _NPAD + k * _CH, _CH)],
                         idx_bufs.at[par], sems[par])

    def drain_idx(par):
        pltpu.make_async_copy(idx_hbm.at[pl.ds(0, _CH)],
                              idx_bufs.at[par], sems[par]).wait()

    @pl.when(jnp.logical_and(cid == 0, sid < _B))
    def _():
        def zb(i, carry):
            for u in range(8):
                acc[pl.ds((i * 8 + u) * 16, 16)] = zeros16
            return carry
        lax.fori_loop(0, _V // 128, zb, None)

        start_idx(sid, 0, 0)
        start_idx(sid, 1, 1)

        def cb(kk, carry):
            for par in range(2):
                k = kk * 2 + par

                @pl.when(k < _NCHUNK)
                def _():
                    drain_idx(par)

                    def gb(g, c3):
                        ivs = [idx_bufs.at[par][pl.ds((g * 10 + u) * 16, 16)]
                               for u in range(10)]
                        for u in range(10):
                            plsc.addupdate_scatter(acc, [ivs[u]], ones16)
                        return c3
                    lax.fori_loop(0, _CH // 160, gb, None)

                    @pl.when(k + 2 < _NCHUNK)
                    def _():
                        start_idx(sid, k + 2, par)
            return carry
        lax.fori_loop(0, (_NCHUNK + 2) // 2, cb, None)

        def rb(i, carry):
            slcs = [pl.ds((i * 4 + u) * 16, 16) for u in range(4)]
            vs = [acc[s] for s in slcs]
            for u in range(4):
                acc[slcs[u]] = 1.0 / jnp.maximum(vs[u], 1.0)
            return carry
        lax.fori_loop(0, _V // 64, rb, None)
        pltpu.sync_copy(acc, recip_hbm.at[pl.ds(sid * _V, _V)])


_counts_call = functools.partial(
    pl.kernel,
    out_type=jax.ShapeDtypeStruct((_B * _V,), jnp.float32),
    mesh=_MESH,
    compiler_params=_SC_PARAMS,
    scratch_types=[
        pltpu.VMEM((_V,), jnp.float32),
        pltpu.VMEM((2, _CH), jnp.int32),
        pltpu.SemaphoreType.DMA,
        pltpu.SemaphoreType.DMA,
    ],
)(_counts_body)


_CHT = 2000


def _transpose_body(ft_hbm, out_hbm, in_bufs, out_buf, sem0, sem1, semo):
    cid = lax.axis_index("c")
    sid = lax.axis_index("s")
    b = sid // 4
    c16 = cid * 64 + (sid % 4) * 16
    sems = (sem0, sem1)
    nchunk = _N // _CHT

    def start_in(k, par):
        pltpu.async_copy(ft_hbm.at[b, pl.ds(k * _CHT, _CHT), pl.ds(c16, 16)],
                         in_bufs.at[par], sems[par])

    def drain_in(par):
        pltpu.make_async_copy(ft_hbm.at[0, pl.ds(0, _CHT), pl.ds(0, 16)],
                              in_bufs.at[par], sems[par]).wait()

    iota16 = lax.iota(jnp.int32, 16)
    cols = [jnp.full((16,), j, jnp.int32) for j in range(16)]

    start_in(0, 0)
    start_in(1, 1)

    def cb(k, carry):
        par_sel = lax.rem(k, 2)
        for par in range(2):
            @pl.when(par_sel == par)
            def _():
                drain_in(par)

                def gb(g, c3):
                    rows = iota16 + g * 16
                    vals = [plsc.load_gather(in_bufs.at[par], [rows, cols[j]])
                            for j in range(16)]
                    for j in range(16):
                        out_buf[j, pl.ds(g * 16, 16)] = vals[j]
                    return c3
                lax.fori_loop(0, _CHT // 16, gb, None)

                @pl.when(k + 2 < nchunk)
                def _():
                    start_in(k + 2, par)
        pltpu.async_copy(
            out_buf,
            out_hbm.at[b, pl.ds(c16, 16), pl.ds(k * _CHT, _CHT)], semo)
        pltpu.make_async_copy(
            out_buf,
            out_hbm.at[0, pl.ds(0, 16), pl.ds(0, _CHT)], semo).wait()
        return carry
    lax.fori_loop(0, nchunk, cb, None)


_transpose_call = functools.partial(
    pl.kernel,
    out_type=jax.ShapeDtypeStruct((_B, _C, _N), jnp.float32),
    mesh=_MESH,
    compiler_params=_SC_PARAMS,
    scratch_types=[
        pltpu.VMEM((2, _CHT, 16), jnp.float32),
        pltpu.VMEM((16, _CHT), jnp.float32),
        pltpu.SemaphoreType.DMA,
        pltpu.SemaphoreType.DMA,
        pltpu.SemaphoreType.DMA,
    ],
)(_transpose_body)


def _scatter_body(feat_hbm, idx_hbm, recip_hbm, out_hbm,
                  acc_a, acc_b, recip_b, idx_bufs, f_a, f_b, sem0, sem1):
    cid = lax.axis_index("c")
    sid = lax.axis_index("s")
    my_batch = sid // 4
    cbase = cid * 64 + (sid % 4) * 16

    zeros16 = jnp.zeros((16,), jnp.float32)
    sems = (sem0, sem1)

    def start_idx(k, par):
        pltpu.async_copy(idx_hbm.at[pl.ds(my_batch * _NPAD + k * _CH, _CH)],
                         idx_bufs.at[par], sems[par])

    def drain_idx(par):
        pltpu.make_async_copy(idx_hbm.at[pl.ds(0, _CH)],
                              idx_bufs.at[par], sems[par]).wait()

    def start_feat(c0, c1, k, par):
        off = k * _CH
        pltpu.async_copy(feat_hbm.at[my_batch, c0, pl.ds(off, _CH)],
                         f_a.at[par], sems[par])
        pltpu.async_copy(feat_hbm.at[my_batch, c1, pl.ds(off, _CH)],
                         f_b.at[par], sems[par])

    def drain_feat(par):
        pltpu.make_async_copy(feat_hbm.at[0, 0, pl.ds(0, _CH)],
                              f_a.at[par], sems[par]).wait()
        pltpu.make_async_copy(feat_hbm.at[0, 0, pl.ds(0, _CH)],
                              f_b.at[par], sems[par]).wait()

    pltpu.sync_copy(recip_hbm.at[pl.ds(my_batch * _V, _V)], recip_b)

    def pair_body(p, carry):
        c0 = cbase + 2 * p
        c1 = c0 + 1

        def zb(i, c2):
            for u in range(8):
                s = pl.ds((i * 8 + u) * 16, 16)
                acc_a[s] = zeros16
                acc_b[s] = zeros16
            return c2
        lax.fori_loop(0, _V // 128, zb, None)

        start_idx(0, 0)
        start_feat(c0, c1, 0, 0)
        start_idx(1, 1)
        start_feat(c0, c1, 1, 1)

        def cb(kk, c2):
            for par in range(2):
                k = kk * 2 + par

                @pl.when(k < _NCHUNK)
                def _():
                    drain_idx(par)
                    drain_feat(par)

                    def gb(g, c3):
                        slcs = [pl.ds((g * 5 + u) * 16, 16) for u in range(5)]
                        ivs = [idx_bufs.at[par][s] for s in slcs]
                        vas = [f_a.at[par][s] for s in slcs]
                        vbs = [f_b.at[par][s] for s in slcs]
                        for u in range(5):
                            plsc.addupdate_scatter(acc_a, [ivs[u]], vas[u])
                            plsc.addupdate_scatter(acc_b, [ivs[u]], vbs[u])
                        return c3
                    lax.fori_loop(0, _CH // 80, gb, None)

                    @pl.when(k + 2 < _NCHUNK)
                    def _():
                        start_idx(k + 2, par)
                        start_feat(c0, c1, k + 2, par)
            return c2
        lax.fori_loop(0, (_NCHUNK + 2) // 2, cb, None)

        def nb(i, c2):
            slcs = [pl.ds((i * 4 + u) * 16, 16) for u in range(4)]
            rs = [recip_b[s] for s in slcs]
            avs = [acc_a[s] for s in slcs]
            bvs = [acc_b[s] for s in slcs]
            for u in range(4):
                acc_a[slcs[u]] = avs[u] * rs[u]
                acc_b[slcs[u]] = bvs[u] * rs[u]
            return c2
        lax.fori_loop(0, _V // 64, nb, None)
        pltpu.sync_copy(acc_a, out_hbm.at[my_batch, c0])
        pltpu.sync_copy(acc_b, out_hbm.at[my_batch, c1])
        return carry
    lax.fori_loop(0, 8, pair_body, None)


_scatter_call = functools.partial(
    pl.kernel,
    out_type=jax.ShapeDtypeStruct((_B, _C, _V), jnp.float32),
    mesh=_MESH,
    compiler_params=_SC_PARAMS,
    scratch_types=[
        pltpu.VMEM((_V,), jnp.float32),
        pltpu.VMEM((_V,), jnp.float32),
        pltpu.VMEM((_V,), jnp.float32),
        pltpu.VMEM((2, _CH), jnp.int32),
        pltpu.VMEM((2, _CH), jnp.float32),
        pltpu.VMEM((2, _CH), jnp.float32),
        pltpu.SemaphoreType.DMA,
        pltpu.SemaphoreType.DMA,
    ],
)(_scatter_body)


def kernel(features, coords):
    nc, idx = _compute_idx(coords)
    recip = _counts_call(idx)
    feat_lin = _transpose_call(jnp.transpose(features, (0, 2, 1)))
    out = _scatter_call(feat_lin, idx, recip)
    return out.reshape(_B, _C, _R, _R, _R), nc


# speedup vs baseline: 2.3603x; 1.0745x over previous
import functools

import jax
import jax.numpy as jnp
from jax import lax
from jax.experimental import pallas as pl
from jax.experimental.pallas import tpu as pltpu
from jax.experimental.pallas import tpu_sc as plsc

_R = 32
_V = _R * _R * _R
_B = 4
_C = 128
_N = 100000
_CH = 4000
_NCHUNK = _N // _CH
_NPAD = 100352

_MESH = plsc.VectorSubcoreMesh(core_axis_name="c", subcore_axis_name="s",
                               num_cores=2, num_subcores=16)
_SC_PARAMS = pltpu.CompilerParams(use_tc_tiling_on_sc=False,
                                  needs_layout_passes=False)


def _coords_body(coords_ref, nc_ref, idx_ref):
    x = coords_ref[0]
    mean = jnp.mean(x, axis=1, keepdims=True)
    c = x - mean
    norm2 = jnp.sum(c * c, axis=0, keepdims=True)
    denom = jnp.max(jnp.sqrt(norm2))
    denom = jnp.maximum(denom * 2.0, 1e-6)
    nc = jnp.clip((c / denom + 0.5) * _R, 0.0, _R - 1.0)
    nc_ref[0] = nc
    vox = jnp.round(nc).astype(jnp.int32)
    idx = vox[0:1] * (_R * _R) + vox[1:2] * _R + vox[2:3]
    idx_ref[pl.ds(0, _N)] = idx.reshape(_N)


def _compute_idx(coords):
    return pl.pallas_call(
        _coords_body,
        grid=(_B,),
        in_specs=[pl.BlockSpec((1, 3, _N), lambda b: (b, 0, 0))],
        out_specs=[
            pl.BlockSpec((1, 3, _N), lambda b: (b, 0, 0)),
            pl.BlockSpec((_NPAD,), lambda b: (b,)),
        ],
        out_shape=[
            jax.ShapeDtypeStruct((_B, 3, _N), jnp.float32),
            jax.ShapeDtypeStruct((_B * _NPAD,), jnp.int32),
        ],
    )(coords)


def _counts_body(idx_hbm, recip_hbm, acc, idx_bufs, sem0, sem1):
    cid = lax.axis_index("c")
    sid = lax.axis_index("s")
    sems = (sem0, sem1)
    zeros16 = jnp.zeros((16,), jnp.float32)
    ones16 = jnp.full((16,), 1.0, jnp.float32)

    def start_idx(batch, k, par):
        pltpu.async_copy(idx_hbm.at[pl.ds(batch * _NPAD + k * _CH, _CH)],
                         idx_bufs.at[par], sems[par])

    def drain_idx(par):
        pltpu.make_async_copy(idx_hbm.at[pl.ds(0, _CH)],
                              idx_bufs.at[par], sems[par]).wait()

    @pl.when(jnp.logical_and(cid == 0, sid < _B))
    def _():
        def zb(i, carry):
            for u in range(8):
                acc[pl.ds((i * 8 + u) * 16, 16)] = zeros16
            return carry
        lax.fori_loop(0, _V // 128, zb, None)

        start_idx(sid, 0, 0)
        start_idx(sid, 1, 1)

        def cb(kk, carry):
            for par in range(2):
                k = kk * 2 + par

                @pl.when(k < _NCHUNK)
                def _():
                    drain_idx(par)

                    def gb(g, c3):
                        ivs = [idx_bufs.at[par][pl.ds((g * 10 + u) * 16, 16)]
                               for u in range(10)]
                        for u in range(10):
                            plsc.addupdate_scatter(acc, [ivs[u]], ones16)
                        return c3
                    lax.fori_loop(0, _CH // 160, gb, None)

                    @pl.when(k + 2 < _NCHUNK)
                    def _():
                        start_idx(sid, k + 2, par)
            return carry
        lax.fori_loop(0, (_NCHUNK + 2) // 2, cb, None)

        def rb(i, carry):
            slcs = [pl.ds((i * 4 + u) * 16, 16) for u in range(4)]
            vs = [acc[s] for s in slcs]
            for u in range(4):
                acc[slcs[u]] = 1.0 / jnp.maximum(vs[u], 1.0)
            return carry
        lax.fori_loop(0, _V // 64, rb, None)
        pltpu.sync_copy(acc, recip_hbm.at[pl.ds(sid * _V, _V)])


_counts_call = functools.partial(
    pl.kernel,
    out_type=jax.ShapeDtypeStruct((_B * _V,), jnp.float32),
    mesh=_MESH,
    compiler_params=_SC_PARAMS,
    scratch_types=[
        pltpu.VMEM((_V,), jnp.float32),
        pltpu.VMEM((2, _CH), jnp.int32),
        pltpu.SemaphoreType.DMA,
        pltpu.SemaphoreType.DMA,
    ],
)(_counts_body)


_CHT = 2000


def _transpose_body(ft_hbm, out_hbm, in_bufs, out_bufs, sem0, sem1, semo):
    cid = lax.axis_index("c")
    sid = lax.axis_index("s")
    b = sid // 4
    c16 = cid * 64 + (sid % 4) * 16
    sems = (sem0, sem1)
    nchunk = _N // _CHT

    def start_in(k, par):
        pltpu.async_copy(ft_hbm.at[b, pl.ds(k * _CHT, _CHT), pl.ds(c16, 16)],
                         in_bufs.at[par], sems[par])

    def drain_in(par):
        pltpu.make_async_copy(ft_hbm.at[0, pl.ds(0, _CHT), pl.ds(0, 16)],
                              in_bufs.at[par], sems[par]).wait()

    iota16 = lax.iota(jnp.int32, 16)
    cols = [jnp.full((16,), j, jnp.int32) for j in range(16)]

    start_in(0, 0)
    start_in(1, 1)

    def cb(kk, carry):
        for par in range(2):
            k = kk * 2 + par
            drain_in(par)

            @pl.when(k >= 2)
            def _():
                pltpu.make_async_copy(
                    out_bufs.at[par],
                    out_hbm.at[0, pl.ds(0, 16), pl.ds(0, _CHT)], semo).wait()

            def gb(g, c3):
                rows = iota16 + g * 16
                vals = [plsc.load_gather(in_bufs.at[par], [rows, cols[j]])
                        for j in range(16)]
                for j in range(16):
                    out_bufs.at[par][j, pl.ds(g * 16, 16)] = vals[j]
                return c3
            lax.fori_loop(0, _CHT // 16, gb, None)

            @pl.when(k + 2 < nchunk)
            def _():
                start_in(k + 2, par)
            pltpu.async_copy(
                out_bufs.at[par],
                out_hbm.at[b, pl.ds(c16, 16), pl.ds(k * _CHT, _CHT)], semo)
        return carry
    lax.fori_loop(0, nchunk // 2, cb, None)
    for par in range(2):
        pltpu.make_async_copy(
            out_bufs.at[par],
            out_hbm.at[0, pl.ds(0, 16), pl.ds(0, _CHT)], semo).wait()


_transpose_call = functools.partial(
    pl.kernel,
    out_type=jax.ShapeDtypeStruct((_B, _C, _N), jnp.float32),
    mesh=_MESH,
    compiler_params=_SC_PARAMS,
    scratch_types=[
        pltpu.VMEM((2, _CHT, 16), jnp.float32),
        pltpu.VMEM((2, 16, _CHT), jnp.float32),
        pltpu.SemaphoreType.DMA,
        pltpu.SemaphoreType.DMA,
        pltpu.SemaphoreType.DMA,
    ],
)(_transpose_body)


def _scatter_body(feat_hbm, idx_hbm, recip_hbm, out_hbm,
                  acc_a, acc_b, recip_b, idx_bufs, f_a, f_b, sem0, sem1):
    cid = lax.axis_index("c")
    sid = lax.axis_index("s")
    my_batch = sid // 4
    cbase = cid * 64 + (sid % 4) * 16

    zeros16 = jnp.zeros((16,), jnp.float32)
    sems = (sem0, sem1)

    def start_idx(k, par):
        pltpu.async_copy(idx_hbm.at[pl.ds(my_batch * _NPAD + k * _CH, _CH)],
                         idx_bufs.at[par], sems[par])

    def drain_idx(par):
        pltpu.make_async_copy(idx_hbm.at[pl.ds(0, _CH)],
                              idx_bufs.at[par], sems[par]).wait()

    def start_feat(c0, c1, k, par):
        off = k * _CH
        pltpu.async_copy(feat_hbm.at[my_batch, c0, pl.ds(off, _CH)],
                         f_a.at[par], sems[par])
        pltpu.async_copy(feat_hbm.at[my_batch, c1, pl.ds(off, _CH)],
                         f_b.at[par], sems[par])

    def drain_feat(par):
        pltpu.make_async_copy(feat_hbm.at[0, 0, pl.ds(0, _CH)],
                              f_a.at[par], sems[par]).wait()
        pltpu.make_async_copy(feat_hbm.at[0, 0, pl.ds(0, _CH)],
                              f_b.at[par], sems[par]).wait()

    pltpu.sync_copy(recip_hbm.at[pl.ds(my_batch * _V, _V)], recip_b)

    def pair_body(p, carry):
        c0 = cbase + 2 * p
        c1 = c0 + 1

        def zb(i, c2):
            for u in range(8):
                s = pl.ds((i * 8 + u) * 16, 16)
                acc_a[s] = zeros16
                acc_b[s] = zeros16
            return c2
        lax.fori_loop(0, _V // 128, zb, None)

        start_idx(0, 0)
        start_feat(c0, c1, 0, 0)
        start_idx(1, 1)
        start_feat(c0, c1, 1, 1)

        def cb(kk, c2):
            for par in range(2):
                k = kk * 2 + par

                @pl.when(k < _NCHUNK)
                def _():
                    drain_idx(par)
                    drain_feat(par)

                    def gb(g, c3):
                        slcs = [pl.ds((g * 5 + u) * 16, 16) for u in range(5)]
                        ivs = [idx_bufs.at[par][s] for s in slcs]
                        vas = [f_a.at[par][s] for s in slcs]
                        vbs = [f_b.at[par][s] for s in slcs]
                        for u in range(5):
                            plsc.addupdate_scatter(acc_a, [ivs[u]], vas[u])
                            plsc.addupdate_scatter(acc_b, [ivs[u]], vbs[u])
                        return c3
                    lax.fori_loop(0, _CH // 80, gb, None)

                    @pl.when(k + 2 < _NCHUNK)
                    def _():
                        start_idx(k + 2, par)
                        start_feat(c0, c1, k + 2, par)
            return c2
        lax.fori_loop(0, (_NCHUNK + 2) // 2, cb, None)

        def nb(i, c2):
            slcs = [pl.ds((i * 4 + u) * 16, 16) for u in range(4)]
            rs = [recip_b[s] for s in slcs]
            avs = [acc_a[s] for s in slcs]
            bvs = [acc_b[s] for s in slcs]
            for u in range(4):
                acc_a[slcs[u]] = avs[u] * rs[u]
                acc_b[slcs[u]] = bvs[u] * rs[u]
            return c2
        lax.fori_loop(0, _V // 64, nb, None)
        pltpu.sync_copy(acc_a, out_hbm.at[my_batch, c0])
        pltpu.sync_copy(acc_b, out_hbm.at[my_batch, c1])
        return carry
    lax.fori_loop(0, 8, pair_body, None)


_scatter_call = functools.partial(
    pl.kernel,
    out_type=jax.ShapeDtypeStruct((_B, _C, _V), jnp.float32),
    mesh=_MESH,
    compiler_params=_SC_PARAMS,
    scratch_types=[
        pltpu.VMEM((_V,), jnp.float32),
        pltpu.VMEM((_V,), jnp.float32),
        pltpu.VMEM((_V,), jnp.float32),
        pltpu.VMEM((2, _CH), jnp.int32),
        pltpu.VMEM((2, _CH), jnp.float32),
        pltpu.VMEM((2, _CH), jnp.float32),
        pltpu.SemaphoreType.DMA,
        pltpu.SemaphoreType.DMA,
    ],
)(_scatter_body)


def kernel(features, coords):
    nc, idx = _compute_idx(coords)
    recip = _counts_call(idx)
    feat_lin = _transpose_call(jnp.transpose(features, (0, 2, 1)))
    out = _scatter_call(feat_lin, idx, recip)
    return out.reshape(_B, _C, _R, _R, _R), nc
